# initial kernel scaffold (unmeasured)
import jax
import jax.numpy as jnp
from jax import lax
from jax.experimental import pallas as pl
from jax.experimental.pallas import tpu as pltpu

N_DEV = 4
SQ = 512
SKV = 2048
HQ = 8
DH = 128
DM = HQ * DH
SCALE = 0.08838834764831843
NEG_INF = -1e30


def _flash_update(slot, q_buf, o_buf, ml_buf, k_ref, v_ref):
    for h in range(HQ):
        q = q_buf[slot, :, h * DH:(h + 1) * DH]
        k = k_ref[h]
        v = v_ref[h]
        s = lax.dot_general(
            q, k, (((1,), (1,)), ((), ())),
            preferred_element_type=jnp.float32,
        ) * SCALE
        m_old = ml_buf[slot, 0, :, h:h + 1]
        l_old = ml_buf[slot, 1, :, h:h + 1]
        mj = jnp.max(s, axis=1, keepdims=True)
        m_new = jnp.maximum(m_old, mj)
        alpha = jnp.exp(m_old - m_new)
        p = jnp.exp(s - m_new)
        l_new = l_old * alpha + jnp.sum(p, axis=1, keepdims=True)
        pv = lax.dot_general(
            p, v, (((1,), (0,)), ((), ())),
            preferred_element_type=jnp.float32,
        )
        o_buf[slot, :, h * DH:(h + 1) * DH] = (
            o_buf[slot, :, h * DH:(h + 1) * DH] * alpha + pv
        )
        ml_buf[slot, 0, :, h:h + 1] = m_new
        ml_buf[slot, 1, :, h:h + 1] = l_new


def _body(x_ref, wq_ref, wo_ref, k_ref, v_ref, out_ref,
          q_buf, o_buf, ml_buf, o_fin, ml_fin,
          q_ssem, q_rsem, o_ssem, o_rsem, ml_ssem, ml_rsem,
          of_ssem, of_rsem, mf_ssem, mf_rsem):
    my = lax.axis_index("i")
    left = lax.rem(my + N_DEV - 1, N_DEV)
    right = lax.rem(my + 1, N_DEV)

    barrier = pltpu.get_barrier_semaphore()
    for nbr in (left, right):
        pl.semaphore_signal(
            barrier, inc=1, device_id=(nbr,),
            device_id_type=pl.DeviceIdType.MESH,
        )
    pl.semaphore_wait(barrier, 2)

    q_buf[0] = jnp.dot(x_ref[...], wq_ref[...],
                       preferred_element_type=jnp.float32)
    o_buf[0] = jnp.zeros((SQ, DM), jnp.float32)
    ml_buf[0, 0] = jnp.full((SQ, HQ), NEG_INF, jnp.float32)
    ml_buf[0, 1] = jnp.zeros((SQ, HQ), jnp.float32)

    for hop in range(N_DEV):
        _flash_update(hop, q_buf, o_buf, ml_buf, k_ref, v_ref)

        if hop < N_DEV - 1:
            rq = pltpu.make_async_remote_copy(
                src_ref=q_buf.at[hop], dst_ref=q_buf.at[hop + 1],
                send_sem=q_ssem.at[hop], recv_sem=q_rsem.at[hop],
                device_id=(right,), device_id_type=pl.DeviceIdType.MESH,
            )
            ro = pltpu.make_async_remote_copy(
                src_ref=o_buf.at[hop], dst_ref=o_buf.at[hop + 1],
                send_sem=o_ssem.at[hop], recv_sem=o_rsem.at[hop],
                device_id=(right,), device_id_type=pl.DeviceIdType.MESH,
            )
            rml = pltpu.make_async_remote_copy(
                src_ref=ml_buf.at[hop], dst_ref=ml_buf.at[hop + 1],
                send_sem=ml_ssem.at[hop], recv_sem=ml_rsem.at[hop],
                device_id=(right,), device_id_type=pl.DeviceIdType.MESH,
            )
            rq.start()
            ro.start()
            rml.start()
            rq.wait()
            ro.wait()
            rml.wait()
        else:
            ro = pltpu.make_async_remote_copy(
                src_ref=o_buf.at[hop], dst_ref=o_fin,
                send_sem=of_ssem, recv_sem=of_rsem,
                device_id=(right,), device_id_type=pl.DeviceIdType.MESH,
            )
            rml = pltpu.make_async_remote_copy(
                src_ref=ml_buf.at[hop], dst_ref=ml_fin,
                send_sem=mf_ssem, recv_sem=mf_rsem,
                device_id=(right,), device_id_type=pl.DeviceIdType.MESH,
            )
            ro.start()
            rml.start()
            ro.wait()
            rml.wait()

    for h in range(HQ):
        o_fin[:, h * DH:(h + 1) * DH] = (
            o_fin[:, h * DH:(h + 1) * DH] / ml_fin[1, :, h:h + 1]
        )
    out_ref[...] = jnp.dot(o_fin[...], wo_ref[...],
                           preferred_element_type=jnp.float32)


def kernel(x, Wq, Wo, K_ext, V_ext):
    xs = x[0]
    K = jnp.transpose(K_ext[0], (1, 0, 2))
    V = jnp.transpose(V_ext[0], (1, 0, 2))

    out = pl.pallas_call(
        _body,
        out_shape=jax.ShapeDtypeStruct((SQ, DM), jnp.float32),
        in_specs=[pl.BlockSpec(memory_space=pltpu.VMEM)] * 5,
        out_specs=pl.BlockSpec(memory_space=pltpu.VMEM),
        scratch_shapes=[
            pltpu.VMEM((N_DEV, SQ, DM), jnp.float32),
            pltpu.VMEM((N_DEV, SQ, DM), jnp.float32),
            pltpu.VMEM((N_DEV, 2, SQ, HQ), jnp.float32),
            pltpu.VMEM((SQ, DM), jnp.float32),
            pltpu.VMEM((2, SQ, HQ), jnp.float32),
            pltpu.SemaphoreType.DMA((N_DEV,)),
            pltpu.SemaphoreType.DMA((N_DEV,)),
            pltpu.SemaphoreType.DMA((N_DEV,)),
            pltpu.SemaphoreType.DMA((N_DEV,)),
            pltpu.SemaphoreType.DMA((N_DEV,)),
            pltpu.SemaphoreType.DMA((N_DEV,)),
            pltpu.SemaphoreType.DMA,
            pltpu.SemaphoreType.DMA,
            pltpu.SemaphoreType.DMA,
            pltpu.SemaphoreType.DMA,
        ],
        compiler_params=pltpu.CompilerParams(collective_id=0),
    )(xs, Wq, Wo, K, V)
    return out[None]


# baseline (device time: 374957 ns/iter reference)
import jax
import jax.numpy as jnp
from jax import lax
from jax.experimental import pallas as pl
from jax.experimental.pallas import tpu as pltpu

N_DEV = 4
SQ = 512
SKV = 2048
HQ = 8
DH = 128
DM = HQ * DH
SCALE = 0.08838834764831843
NEG_INF = -1e30


CKV = 512


def _flash_update(slot, q_buf, o_buf, ml_buf, k_ref, v_ref):
    for h in range(HQ):
        q = q_buf[slot, :, h * DH:(h + 1) * DH]
        for c in range(SKV // CKV):
            k = k_ref[h, c * CKV:(c + 1) * CKV, :]
            v = v_ref[h, c * CKV:(c + 1) * CKV, :]
            s = lax.dot_general(
                q, k, (((1,), (1,)), ((), ())),
                preferred_element_type=jnp.float32,
            ) * SCALE
            m_old = ml_buf[slot, 0, :, h:h + 1]
            l_old = ml_buf[slot, 1, :, h:h + 1]
            mj = jnp.max(s, axis=1, keepdims=True)
            m_new = jnp.maximum(m_old, mj)
            alpha = jnp.exp(m_old - m_new)
            p = jnp.exp(s - m_new)
            l_new = l_old * alpha + jnp.sum(p, axis=1, keepdims=True)
            pv = lax.dot_general(
                p, v, (((1,), (0,)), ((), ())),
                preferred_element_type=jnp.float32,
            )
            o_buf[slot, :, h * DH:(h + 1) * DH] = (
                o_buf[slot, :, h * DH:(h + 1) * DH] * alpha + pv
            )
            ml_buf[slot, 0, :, h:h + 1] = m_new
            ml_buf[slot, 1, :, h:h + 1] = l_new


def _body(x_ref, wq_ref, wo_ref, k_ref, v_ref, out_ref,
          q_buf, o_buf, ml_buf,
          q_ssem, q_rsem, o_ssem, o_rsem, ml_ssem, ml_rsem,
          of_ssem, of_rsem, mf_ssem, mf_rsem):
    my = lax.axis_index("i")
    left = lax.rem(my + N_DEV - 1, N_DEV)
    right = lax.rem(my + 1, N_DEV)

    barrier = pltpu.get_barrier_semaphore()
    for nbr in (left, right):
        pl.semaphore_signal(
            barrier, inc=1, device_id=(nbr,),
            device_id_type=pl.DeviceIdType.MESH,
        )
    pl.semaphore_wait(barrier, 2)

    q_buf[0] = jnp.dot(x_ref[...], wq_ref[...],
                       preferred_element_type=jnp.float32)
    o_buf[0] = jnp.zeros((SQ, DM), jnp.float32)
    ml_buf[0, 0] = jnp.full((SQ, HQ), NEG_INF, jnp.float32)
    ml_buf[0, 1] = jnp.zeros((SQ, HQ), jnp.float32)

    for hop in range(N_DEV):
        _flash_update(hop, q_buf, o_buf, ml_buf, k_ref, v_ref)

        if hop < N_DEV - 1:
            rq = pltpu.make_async_remote_copy(
                src_ref=q_buf.at[hop], dst_ref=q_buf.at[hop + 1],
                send_sem=q_ssem.at[hop], recv_sem=q_rsem.at[hop],
                device_id=(right,), device_id_type=pl.DeviceIdType.MESH,
            )
            ro = pltpu.make_async_remote_copy(
                src_ref=o_buf.at[hop], dst_ref=o_buf.at[hop + 1],
                send_sem=o_ssem.at[hop], recv_sem=o_rsem.at[hop],
                device_id=(right,), device_id_type=pl.DeviceIdType.MESH,
            )
            rml = pltpu.make_async_remote_copy(
                src_ref=ml_buf.at[hop], dst_ref=ml_buf.at[hop + 1],
                send_sem=ml_ssem.at[hop], recv_sem=ml_rsem.at[hop],
                device_id=(right,), device_id_type=pl.DeviceIdType.MESH,
            )
            rq.start()
            ro.start()
            rml.start()
            rq.wait()
            ro.wait()
            rml.wait()
        else:
            ro = pltpu.make_async_remote_copy(
                src_ref=o_buf.at[hop], dst_ref=o_buf.at[0],
                send_sem=of_ssem, recv_sem=of_rsem,
                device_id=(right,), device_id_type=pl.DeviceIdType.MESH,
            )
            rml = pltpu.make_async_remote_copy(
                src_ref=ml_buf.at[hop], dst_ref=ml_buf.at[0],
                send_sem=mf_ssem, recv_sem=mf_rsem,
                device_id=(right,), device_id_type=pl.DeviceIdType.MESH,
            )
            ro.start()
            rml.start()
            ro.wait()
            rml.wait()

    for h in range(HQ):
        o_buf[0, :, h * DH:(h + 1) * DH] = (
            o_buf[0, :, h * DH:(h + 1) * DH] / ml_buf[0, 1, :, h:h + 1]
        )
    out_ref[...] = jnp.dot(o_buf[0], wo_ref[...],
                           preferred_element_type=jnp.float32)


def kernel(x, Wq, Wo, K_ext, V_ext):
    xs = x[0]
    K = jnp.transpose(K_ext[0], (1, 0, 2))
    V = jnp.transpose(V_ext[0], (1, 0, 2))

    out = pl.pallas_call(
        _body,
        out_shape=jax.ShapeDtypeStruct((SQ, DM), jnp.float32),
        in_specs=[pl.BlockSpec(memory_space=pltpu.VMEM)] * 5,
        out_specs=pl.BlockSpec(memory_space=pltpu.VMEM),
        scratch_shapes=[
            pltpu.VMEM((N_DEV, SQ, DM), jnp.float32),
            pltpu.VMEM((N_DEV, SQ, DM), jnp.float32),
            pltpu.VMEM((N_DEV, 2, SQ, HQ), jnp.float32),
            pltpu.SemaphoreType.DMA((N_DEV,)),
            pltpu.SemaphoreType.DMA((N_DEV,)),
            pltpu.SemaphoreType.DMA((N_DEV,)),
            pltpu.SemaphoreType.DMA((N_DEV,)),
            pltpu.SemaphoreType.DMA((N_DEV,)),
            pltpu.SemaphoreType.DMA((N_DEV,)),
            pltpu.SemaphoreType.DMA,
            pltpu.SemaphoreType.DMA,
            pltpu.SemaphoreType.DMA,
            pltpu.SemaphoreType.DMA,
        ],
        compiler_params=pltpu.CompilerParams(
            collective_id=0,
            vmem_limit_bytes=62 * 1024 * 1024,
        ),
    )(xs, Wq, Wo, K, V)
    return out[None]


# device time: 231154 ns/iter; 1.6221x vs baseline; 1.6221x over previous
import jax
import jax.numpy as jnp
from jax import lax
from jax.experimental import pallas as pl
from jax.experimental.pallas import tpu as pltpu

N_DEV = 4
SQ = 512
SKV = 2048
HQ = 8
DH = 128
DM = HQ * DH
SCALE = 0.08838834764831843
CKV = 512


def _local_partial(slot, q_buf, o_ref, ml_ref, k_ref, v_ref):
    for h in range(HQ):
        q = q_buf[slot, :, h * DH:(h + 1) * DH]
        for c in range(SKV // CKV):
            k = k_ref[h, c * CKV:(c + 1) * CKV, :]
            v = v_ref[h, c * CKV:(c + 1) * CKV, :]
            s = lax.dot_general(
                q, k, (((1,), (1,)), ((), ())),
                preferred_element_type=jnp.float32,
            ) * SCALE
            mj = jnp.max(s, axis=1, keepdims=True)
            if c == 0:
                m_new = mj
                alpha = None
            else:
                m_old = ml_ref[0, :, h:h + 1]
                m_new = jnp.maximum(m_old, mj)
                alpha = jnp.exp(m_old - m_new)
            p = jnp.exp(s - m_new)
            pv = lax.dot_general(
                p, v, (((1,), (0,)), ((), ())),
                preferred_element_type=jnp.float32,
            )
            psum = jnp.sum(p, axis=1, keepdims=True)
            if c == 0:
                o_ref[:, h * DH:(h + 1) * DH] = pv
                ml_ref[1, :, h:h + 1] = psum
            else:
                o_ref[:, h * DH:(h + 1) * DH] = (
                    o_ref[:, h * DH:(h + 1) * DH] * alpha + pv
                )
                ml_ref[1, :, h:h + 1] = ml_ref[1, :, h:h + 1] * alpha + psum
            ml_ref[0, :, h:h + 1] = m_new


def _body(x_ref, wq_ref, wo_ref, k_ref, v_ref, out_ref,
          q_buf, o_buf, ml_buf, o_loc, ml_loc,
          q_ssem, q_rsem, o_ssem, o_rsem, ml_ssem, ml_rsem):
    my = lax.axis_index("i")
    left = lax.rem(my + N_DEV - 1, N_DEV)
    right = lax.rem(my + 1, N_DEV)

    def acc_copy(hop, dst_slot):
        ro = pltpu.make_async_remote_copy(
            src_ref=o_buf.at[hop], dst_ref=o_buf.at[dst_slot],
            send_sem=o_ssem.at[hop], recv_sem=o_rsem.at[hop],
            device_id=(right,), device_id_type=pl.DeviceIdType.MESH,
        )
        rml = pltpu.make_async_remote_copy(
            src_ref=ml_buf.at[hop], dst_ref=ml_buf.at[dst_slot],
            send_sem=ml_ssem.at[hop], recv_sem=ml_rsem.at[hop],
            device_id=(right,), device_id_type=pl.DeviceIdType.MESH,
        )
        return ro, rml

    def q_copy(hop):
        return pltpu.make_async_remote_copy(
            src_ref=q_buf.at[hop], dst_ref=q_buf.at[hop + 1],
            send_sem=q_ssem.at[hop], recv_sem=q_rsem.at[hop],
            device_id=(right,), device_id_type=pl.DeviceIdType.MESH,
        )

    barrier = pltpu.get_barrier_semaphore()
    for nbr in (left, right):
        pl.semaphore_signal(
            barrier, inc=1, device_id=(nbr,),
            device_id_type=pl.DeviceIdType.MESH,
        )
    pl.semaphore_wait(barrier, 2)

    pending_sends = []

    q_buf[0] = jnp.dot(x_ref[...], wq_ref[...],
                       preferred_element_type=jnp.float32)
    rq0 = q_copy(0)
    rq0.start()
    pending_sends.append(rq0)

    for hop in range(N_DEV):
        if hop > 0:
            q_copy(hop - 1).wait_recv()
            if hop < N_DEV - 1:
                rq = q_copy(hop)
                rq.start()
                pending_sends.append(rq)

        if hop == 0:
            _local_partial(0, q_buf, o_buf.at[0], ml_buf.at[0],
                           k_ref, v_ref)
        else:
            _local_partial(hop, q_buf, o_loc, ml_loc, k_ref, v_ref)

            ro_in, rml_in = acc_copy(hop - 1, hop)
            ro_in.wait_recv()
            rml_in.wait_recv()
            m_in = ml_buf[hop, 0]
            l_in = ml_buf[hop, 1]
            m_lc = ml_loc[0]
            l_lc = ml_loc[1]
            m_new = jnp.maximum(m_in, m_lc)
            a_in = jnp.exp(m_in - m_new)
            a_lc = jnp.exp(m_lc - m_new)
            ml_buf[hop, 0] = m_new
            ml_buf[hop, 1] = l_in * a_in + l_lc * a_lc
            for h in range(HQ):
                o_buf[hop, :, h * DH:(h + 1) * DH] = (
                    o_buf[hop, :, h * DH:(h + 1) * DH] * a_in[:, h:h + 1]
                    + o_loc[:, h * DH:(h + 1) * DH] * a_lc[:, h:h + 1]
                )

        dst_slot = hop + 1 if hop < N_DEV - 1 else 0
        ro, rml = acc_copy(hop, dst_slot)
        ro.start()
        rml.start()
        pending_sends.append(ro)
        pending_sends.append(rml)

    ro_fin, rml_fin = acc_copy(N_DEV - 1, 0)
    ro_fin.wait_recv()
    rml_fin.wait_recv()

    for h in range(HQ):
        o_buf[0, :, h * DH:(h + 1) * DH] = (
            o_buf[0, :, h * DH:(h + 1) * DH] / ml_buf[0, 1, :, h:h + 1]
        )
    out_ref[...] = jnp.dot(o_buf[0], wo_ref[...],
                           preferred_element_type=jnp.float32)

    for r in pending_sends:
        r.wait_send()


def kernel(x, Wq, Wo, K_ext, V_ext):
    xs = x[0]
    K = jnp.transpose(K_ext[0], (1, 0, 2))
    V = jnp.transpose(V_ext[0], (1, 0, 2))

    out = pl.pallas_call(
        _body,
        out_shape=jax.ShapeDtypeStruct((SQ, DM), jnp.float32),
        in_specs=[pl.BlockSpec(memory_space=pltpu.VMEM)] * 5,
        out_specs=pl.BlockSpec(memory_space=pltpu.VMEM),
        scratch_shapes=[
            pltpu.VMEM((N_DEV, SQ, DM), jnp.float32),
            pltpu.VMEM((N_DEV, SQ, DM), jnp.float32),
            pltpu.VMEM((N_DEV, 2, SQ, HQ), jnp.float32),
            pltpu.VMEM((SQ, DM), jnp.float32),
            pltpu.VMEM((2, SQ, HQ), jnp.float32),
            pltpu.SemaphoreType.DMA((N_DEV,)),
            pltpu.SemaphoreType.DMA((N_DEV,)),
            pltpu.SemaphoreType.DMA((N_DEV,)),
            pltpu.SemaphoreType.DMA((N_DEV,)),
            pltpu.SemaphoreType.DMA((N_DEV,)),
            pltpu.SemaphoreType.DMA((N_DEV,)),
        ],
        compiler_params=pltpu.CompilerParams(
            collective_id=0,
            vmem_limit_bytes=62 * 1024 * 1024,
        ),
    )(xs, Wq, Wo, K, V)
    return out[None]


# device time: 204571 ns/iter; 1.8329x vs baseline; 1.1299x over previous
import jax
import jax.numpy as jnp
from jax import lax
from jax.experimental import pallas as pl
from jax.experimental.pallas import tpu as pltpu

N_DEV = 4
SQ = 512
SKV = 2048
HQ = 8
DH = 128
DM = HQ * DH
SCALE = 0.08838834764831843
CKV = 512


def _local_partial(slot, q_buf, o_ref, ml_ref, k_ref, v_ref):
    for h in range(HQ):
        q = q_buf[slot, :, h * DH:(h + 1) * DH]
        for c in range(SKV // CKV):
            k = k_ref[h, c * CKV:(c + 1) * CKV, :]
            v = v_ref[h, c * CKV:(c + 1) * CKV, :]
            s = lax.dot_general(
                q, k, (((1,), (1,)), ((), ())),
                preferred_element_type=jnp.float32,
            ) * SCALE
            mj = jnp.max(s, axis=1, keepdims=True)
            if c == 0:
                m_new = mj
                alpha = None
            else:
                m_old = ml_ref[0, :, h:h + 1]
                m_new = jnp.maximum(m_old, mj)
                alpha = jnp.exp(m_old - m_new)
            p = jnp.exp(s - m_new)
            pv = lax.dot_general(
                p.astype(jnp.bfloat16), v, (((1,), (0,)), ((), ())),
                preferred_element_type=jnp.float32,
            )
            psum = jnp.sum(p, axis=1, keepdims=True)
            if c == 0:
                o_ref[:, h * DH:(h + 1) * DH] = pv
                ml_ref[1, :, h:h + 1] = psum
            else:
                o_ref[:, h * DH:(h + 1) * DH] = (
                    o_ref[:, h * DH:(h + 1) * DH] * alpha + pv
                )
                ml_ref[1, :, h:h + 1] = ml_ref[1, :, h:h + 1] * alpha + psum
            ml_ref[0, :, h:h + 1] = m_new


def _body(x_ref, wq_ref, wo_ref, k_ref, v_ref, out_ref,
          q_buf, o_buf, ml_buf, o_loc, ml_loc,
          q_ssem, q_rsem, o_ssem, o_rsem, ml_ssem, ml_rsem):
    my = lax.axis_index("i")
    left = lax.rem(my + N_DEV - 1, N_DEV)
    right = lax.rem(my + 1, N_DEV)

    def acc_copy(hop, dst_slot):
        ro = pltpu.make_async_remote_copy(
            src_ref=o_buf.at[hop], dst_ref=o_buf.at[dst_slot],
            send_sem=o_ssem.at[hop], recv_sem=o_rsem.at[hop],
            device_id=(right,), device_id_type=pl.DeviceIdType.MESH,
        )
        rml = pltpu.make_async_remote_copy(
            src_ref=ml_buf.at[hop], dst_ref=ml_buf.at[dst_slot],
            send_sem=ml_ssem.at[hop], recv_sem=ml_rsem.at[hop],
            device_id=(right,), device_id_type=pl.DeviceIdType.MESH,
        )
        return ro, rml

    def q_copy(hop):
        return pltpu.make_async_remote_copy(
            src_ref=q_buf.at[hop], dst_ref=q_buf.at[hop + 1],
            send_sem=q_ssem.at[hop], recv_sem=q_rsem.at[hop],
            device_id=(right,), device_id_type=pl.DeviceIdType.MESH,
        )

    barrier = pltpu.get_barrier_semaphore()
    for nbr in (left, right):
        pl.semaphore_signal(
            barrier, inc=1, device_id=(nbr,),
            device_id_type=pl.DeviceIdType.MESH,
        )
    pl.semaphore_wait(barrier, 2)

    pending_sends = []

    q_buf[0] = jnp.dot(x_ref[...], wq_ref[...],
                       preferred_element_type=jnp.float32
                       ).astype(jnp.bfloat16)
    rq0 = q_copy(0)
    rq0.start()
    pending_sends.append(rq0)

    for hop in range(N_DEV):
        if hop > 0:
            q_copy(hop - 1).wait_recv()
            if hop < N_DEV - 1:
                rq = q_copy(hop)
                rq.start()
                pending_sends.append(rq)

        if hop == 0:
            _local_partial(0, q_buf, o_buf.at[0], ml_buf.at[0],
                           k_ref, v_ref)
        else:
            _local_partial(hop, q_buf, o_loc, ml_loc, k_ref, v_ref)

            ro_in, rml_in = acc_copy(hop - 1, hop)
            ro_in.wait_recv()
            rml_in.wait_recv()
            m_in = ml_buf[hop, 0]
            l_in = ml_buf[hop, 1]
            m_lc = ml_loc[0]
            l_lc = ml_loc[1]
            m_new = jnp.maximum(m_in, m_lc)
            a_in = jnp.exp(m_in - m_new)
            a_lc = jnp.exp(m_lc - m_new)
            ml_buf[hop, 0] = m_new
            ml_buf[hop, 1] = l_in * a_in + l_lc * a_lc
            for h in range(HQ):
                o_buf[hop, :, h * DH:(h + 1) * DH] = (
                    o_buf[hop, :, h * DH:(h + 1) * DH] * a_in[:, h:h + 1]
                    + o_loc[:, h * DH:(h + 1) * DH] * a_lc[:, h:h + 1]
                )

        dst_slot = hop + 1 if hop < N_DEV - 1 else 0
        ro, rml = acc_copy(hop, dst_slot)
        ro.start()
        rml.start()
        pending_sends.append(ro)
        pending_sends.append(rml)

    ro_fin, rml_fin = acc_copy(N_DEV - 1, 0)
    ro_fin.wait_recv()
    rml_fin.wait_recv()

    for h in range(HQ):
        o_buf[0, :, h * DH:(h + 1) * DH] = (
            o_buf[0, :, h * DH:(h + 1) * DH] / ml_buf[0, 1, :, h:h + 1]
        )
    out_ref[...] = jnp.dot(o_buf[0], wo_ref[...],
                           preferred_element_type=jnp.float32)

    for r in pending_sends:
        r.wait_send()


def kernel(x, Wq, Wo, K_ext, V_ext):
    xs = x[0]
    K = jnp.transpose(K_ext[0], (1, 0, 2)).astype(jnp.bfloat16)
    V = jnp.transpose(V_ext[0], (1, 0, 2)).astype(jnp.bfloat16)

    out = pl.pallas_call(
        _body,
        out_shape=jax.ShapeDtypeStruct((SQ, DM), jnp.float32),
        in_specs=[pl.BlockSpec(memory_space=pltpu.VMEM)] * 5,
        out_specs=pl.BlockSpec(memory_space=pltpu.VMEM),
        scratch_shapes=[
            pltpu.VMEM((N_DEV, SQ, DM), jnp.bfloat16),
            pltpu.VMEM((N_DEV, SQ, DM), jnp.float32),
            pltpu.VMEM((N_DEV, 2, SQ, HQ), jnp.float32),
            pltpu.VMEM((SQ, DM), jnp.float32),
            pltpu.VMEM((2, SQ, HQ), jnp.float32),
            pltpu.SemaphoreType.DMA((N_DEV,)),
            pltpu.SemaphoreType.DMA((N_DEV,)),
            pltpu.SemaphoreType.DMA((N_DEV,)),
            pltpu.SemaphoreType.DMA((N_DEV,)),
            pltpu.SemaphoreType.DMA((N_DEV,)),
            pltpu.SemaphoreType.DMA((N_DEV,)),
        ],
        compiler_params=pltpu.CompilerParams(
            collective_id=0,
            vmem_limit_bytes=62 * 1024 * 1024,
        ),
    )(xs, Wq, Wo, K, V)
    return out[None]


# device time: 186162 ns/iter; 2.0141x vs baseline; 1.0989x over previous
import jax
import jax.numpy as jnp
from jax import lax
from jax.experimental import pallas as pl
from jax.experimental.pallas import tpu as pltpu

N_DEV = 4
SQ = 512
SKV = 2048
HQ = 8
DH = 128
DM = HQ * DH
SCALE = 0.08838834764831843
CKV = 512


def _local_partial(slot, q_buf, o_ref, ml_ref, k_ref, v_ref):
    for h in range(HQ):
        q = q_buf[slot, :, h * DH:(h + 1) * DH]
        for c in range(SKV // CKV):
            k = k_ref[h, c * CKV:(c + 1) * CKV, :]
            v = v_ref[h, c * CKV:(c + 1) * CKV, :]
            s = lax.dot_general(
                q, k, (((1,), (1,)), ((), ())),
                preferred_element_type=jnp.float32,
            ) * SCALE
            mj = jnp.max(s, axis=1, keepdims=True)
            if c == 0:
                m_new = mj
                alpha = None
            else:
                m_old = ml_ref[0, :, h:h + 1]
                m_new = jnp.maximum(m_old, mj)
                alpha = jnp.exp(m_old - m_new)
            p = jnp.exp(s - m_new)
            pv = lax.dot_general(
                p.astype(jnp.bfloat16), v, (((1,), (0,)), ((), ())),
                preferred_element_type=jnp.float32,
            )
            psum = jnp.sum(p, axis=1, keepdims=True)
            if c == 0:
                o_ref[:, h * DH:(h + 1) * DH] = pv
                ml_ref[1, :, h:h + 1] = psum
            else:
                o_ref[:, h * DH:(h + 1) * DH] = (
                    o_ref[:, h * DH:(h + 1) * DH] * alpha + pv
                )
                ml_ref[1, :, h:h + 1] = ml_ref[1, :, h:h + 1] * alpha + psum
            ml_ref[0, :, h:h + 1] = m_new


def _body(x_ref, wq_ref, wo_ref, k_ref, v_ref, out_ref,
          q_buf, psend, collect, ml_send, ml_coll,
          q_ssem, q_rsem, o_ssem, o_rsem, ml_ssem, ml_rsem):
    my = lax.axis_index("i")
    left = lax.rem(my + N_DEV - 1, N_DEV)
    right = lax.rem(my + 1, N_DEV)
    diag = lax.rem(my + 2, N_DEV)

    def q_copy(hop):
        return pltpu.make_async_remote_copy(
            src_ref=q_buf.at[hop], dst_ref=q_buf.at[hop + 1],
            send_sem=q_ssem.at[hop], recv_sem=q_rsem.at[hop],
            device_id=(right,), device_id_type=pl.DeviceIdType.MESH,
        )

    def partial_copy(hop, target):
        ro = pltpu.make_async_remote_copy(
            src_ref=psend.at[hop], dst_ref=collect.at[hop],
            send_sem=o_ssem.at[hop], recv_sem=o_rsem.at[hop],
            device_id=(target,), device_id_type=pl.DeviceIdType.MESH,
        )
        rml = pltpu.make_async_remote_copy(
            src_ref=ml_send.at[hop], dst_ref=ml_coll.at[hop],
            send_sem=ml_ssem.at[hop], recv_sem=ml_rsem.at[hop],
            device_id=(target,), device_id_type=pl.DeviceIdType.MESH,
        )
        return ro, rml

    barrier = pltpu.get_barrier_semaphore()
    for nbr in (left, right, diag):
        pl.semaphore_signal(
            barrier, inc=1, device_id=(nbr,),
            device_id_type=pl.DeviceIdType.MESH,
        )
    pl.semaphore_wait(barrier, 3)

    pending_sends = []

    q_buf[0] = jnp.dot(x_ref[...], wq_ref[...],
                       preferred_element_type=jnp.float32
                       ).astype(jnp.bfloat16)
    rq0 = q_copy(0)
    rq0.start()
    pending_sends.append(rq0)

    for hop in range(N_DEV):
        if hop > 0:
            q_copy(hop - 1).wait_recv()
            if hop < N_DEV - 1:
                rq = q_copy(hop)
                rq.start()
                pending_sends.append(rq)

        if hop == 0:
            _local_partial(0, q_buf, collect.at[0], ml_coll.at[0],
                           k_ref, v_ref)
        else:
            _local_partial(hop, q_buf, psend.at[hop], ml_send.at[hop],
                           k_ref, v_ref)
            home = lax.rem(my + N_DEV - hop, N_DEV)
            ro, rml = partial_copy(hop, home)
            ro.start()
            rml.start()
            pending_sends.append(ro)
            pending_sends.append(rml)

    for j in range(1, N_DEV):
        ro_in, rml_in = partial_copy(j, my)
        ro_in.wait_recv()
        rml_in.wait_recv()
        m0 = ml_coll[0, 0]
        l0 = ml_coll[0, 1]
        mj = ml_coll[j, 0]
        lj = ml_coll[j, 1]
        m_new = jnp.maximum(m0, mj)
        a0 = jnp.exp(m0 - m_new)
        aj = jnp.exp(mj - m_new)
        ml_coll[0, 0] = m_new
        ml_coll[0, 1] = l0 * a0 + lj * aj
        for h in range(HQ):
            collect[0, :, h * DH:(h + 1) * DH] = (
                collect[0, :, h * DH:(h + 1) * DH] * a0[:, h:h + 1]
                + collect[j, :, h * DH:(h + 1) * DH] * aj[:, h:h + 1]
            )

    for h in range(HQ):
        collect[0, :, h * DH:(h + 1) * DH] = (
            collect[0, :, h * DH:(h + 1) * DH] / ml_coll[0, 1, :, h:h + 1]
        )
    out_ref[...] = jnp.dot(collect[0], wo_ref[...],
                           preferred_element_type=jnp.float32)

    for r in pending_sends:
        r.wait_send()


def kernel(x, Wq, Wo, K_ext, V_ext):
    xs = x[0]
    K = jnp.transpose(K_ext[0], (1, 0, 2)).astype(jnp.bfloat16)
    V = jnp.transpose(V_ext[0], (1, 0, 2)).astype(jnp.bfloat16)

    out = pl.pallas_call(
        _body,
        out_shape=jax.ShapeDtypeStruct((SQ, DM), jnp.float32),
        in_specs=[pl.BlockSpec(memory_space=pltpu.VMEM)] * 5,
        out_specs=pl.BlockSpec(memory_space=pltpu.VMEM),
        scratch_shapes=[
            pltpu.VMEM((N_DEV, SQ, DM), jnp.bfloat16),
            pltpu.VMEM((N_DEV, SQ, DM), jnp.float32),
            pltpu.VMEM((N_DEV, SQ, DM), jnp.float32),
            pltpu.VMEM((N_DEV, 2, SQ, HQ), jnp.float32),
            pltpu.VMEM((N_DEV, 2, SQ, HQ), jnp.float32),
            pltpu.SemaphoreType.DMA((N_DEV,)),
            pltpu.SemaphoreType.DMA((N_DEV,)),
            pltpu.SemaphoreType.DMA((N_DEV,)),
            pltpu.SemaphoreType.DMA((N_DEV,)),
            pltpu.SemaphoreType.DMA((N_DEV,)),
            pltpu.SemaphoreType.DMA((N_DEV,)),
        ],
        compiler_params=pltpu.CompilerParams(
            collective_id=0,
            vmem_limit_bytes=62 * 1024 * 1024,
        ),
    )(xs, Wq, Wo, K, V)
    return out[None]


# device time: 106418 ns/iter; 3.5234x vs baseline; 1.7493x over previous
import jax
import jax.numpy as jnp
from jax import lax
from jax.experimental import pallas as pl
from jax.experimental.pallas import tpu as pltpu

N_DEV = 4
SQ = 512
SKV = 2048
HQ = 8
DH = 128
DM = HQ * DH
SCALE = 0.08838834764831843
LOG2E = 1.4426950408889634
S2 = SCALE * LOG2E


def _local_partial(slot, q_buf, o_ref, ml_ref, k_ref, v_ref):
    for h in range(HQ):
        q = q_buf[slot, :, h * DH:(h + 1) * DH]
        k = k_ref[h]
        v = v_ref[h]
        s = lax.dot_general(
            q, k, (((1,), (1,)), ((), ())),
            preferred_element_type=jnp.float32,
        ) * S2
        m = jnp.max(s, axis=1, keepdims=True)
        p = jnp.exp2(s - m)
        pv = lax.dot_general(
            p.astype(jnp.bfloat16), v, (((1,), (0,)), ((), ())),
            preferred_element_type=jnp.float32,
        )
        o_ref[:, h * DH:(h + 1) * DH] = pv.astype(jnp.bfloat16)
        ml_ref[0, :, h:h + 1] = m
        ml_ref[1, :, h:h + 1] = jnp.sum(p, axis=1, keepdims=True)


def _body(x_ref, wq_ref, wo_ref, k_ref, v_ref, out_ref,
          q_buf, psend, collect, ml_send, ml_coll, onorm,
          q_ssem, q_rsem, o_ssem, o_rsem, ml_ssem, ml_rsem):
    my = lax.axis_index("i")
    left = lax.rem(my + N_DEV - 1, N_DEV)
    right = lax.rem(my + 1, N_DEV)
    diag = lax.rem(my + 2, N_DEV)

    def q_copy(hop):
        return pltpu.make_async_remote_copy(
            src_ref=q_buf.at[hop], dst_ref=q_buf.at[hop + 1],
            send_sem=q_ssem.at[hop], recv_sem=q_rsem.at[hop],
            device_id=(right,), device_id_type=pl.DeviceIdType.MESH,
        )

    def partial_copy(hop, target):
        ro = pltpu.make_async_remote_copy(
            src_ref=psend.at[hop], dst_ref=collect.at[hop],
            send_sem=o_ssem.at[hop], recv_sem=o_rsem.at[hop],
            device_id=(target,), device_id_type=pl.DeviceIdType.MESH,
        )
        rml = pltpu.make_async_remote_copy(
            src_ref=ml_send.at[hop], dst_ref=ml_coll.at[hop],
            send_sem=ml_ssem.at[hop], recv_sem=ml_rsem.at[hop],
            device_id=(target,), device_id_type=pl.DeviceIdType.MESH,
        )
        return ro, rml

    barrier = pltpu.get_barrier_semaphore()
    for nbr in (left, right, diag):
        pl.semaphore_signal(
            barrier, inc=1, device_id=(nbr,),
            device_id_type=pl.DeviceIdType.MESH,
        )
    pl.semaphore_wait(barrier, 3)

    pending_sends = []

    q_buf[0] = jnp.dot(x_ref[...], wq_ref[...],
                       preferred_element_type=jnp.float32
                       ).astype(jnp.bfloat16)
    rq0 = q_copy(0)
    rq0.start()
    pending_sends.append(rq0)

    for hop in range(N_DEV):
        if hop > 0:
            q_copy(hop - 1).wait_recv()
            if hop < N_DEV - 1:
                rq = q_copy(hop)
                rq.start()
                pending_sends.append(rq)

        if hop == 0:
            _local_partial(0, q_buf, collect.at[0], ml_coll.at[0],
                           k_ref, v_ref)
        else:
            _local_partial(hop, q_buf, psend.at[hop], ml_send.at[hop],
                           k_ref, v_ref)
            home = lax.rem(my + N_DEV - hop, N_DEV)
            ro, rml = partial_copy(hop, home)
            ro.start()
            rml.start()
            pending_sends.append(ro)
            pending_sends.append(rml)

    for j in range(1, N_DEV):
        ro_in, rml_in = partial_copy(j, my)
        ro_in.wait_recv()
        rml_in.wait_recv()
        m0 = ml_coll[0, 0]
        l0 = ml_coll[0, 1]
        mj = ml_coll[j, 0]
        lj = ml_coll[j, 1]
        m_new = jnp.maximum(m0, mj)
        a0 = jnp.exp2(m0 - m_new)
        aj = jnp.exp2(mj - m_new)
        ml_coll[0, 0] = m_new
        ml_coll[0, 1] = l0 * a0 + lj * aj
        for h in range(HQ):
            collect[0, :, h * DH:(h + 1) * DH] = (
                collect[0, :, h * DH:(h + 1) * DH] * a0[:, h:h + 1]
                + collect[j, :, h * DH:(h + 1) * DH] * aj[:, h:h + 1]
            ).astype(jnp.bfloat16)

    for h in range(HQ):
        onorm[:, h * DH:(h + 1) * DH] = (
            collect[0, :, h * DH:(h + 1) * DH] / ml_coll[0, 1, :, h:h + 1]
        )
    out_ref[...] = jnp.dot(onorm[...], wo_ref[...],
                           preferred_element_type=jnp.float32)

    for r in pending_sends:
        r.wait_send()


def kernel(x, Wq, Wo, K_ext, V_ext):
    xs = x[0]
    K = jnp.transpose(K_ext[0], (1, 0, 2)).astype(jnp.bfloat16)
    V = jnp.transpose(V_ext[0], (1, 0, 2)).astype(jnp.bfloat16)

    out = pl.pallas_call(
        _body,
        out_shape=jax.ShapeDtypeStruct((SQ, DM), jnp.float32),
        in_specs=[pl.BlockSpec(memory_space=pltpu.VMEM)] * 5,
        out_specs=pl.BlockSpec(memory_space=pltpu.VMEM),
        scratch_shapes=[
            pltpu.VMEM((N_DEV, SQ, DM), jnp.bfloat16),
            pltpu.VMEM((N_DEV, SQ, DM), jnp.bfloat16),
            pltpu.VMEM((N_DEV, SQ, DM), jnp.bfloat16),
            pltpu.VMEM((N_DEV, 2, SQ, HQ), jnp.float32),
            pltpu.VMEM((N_DEV, 2, SQ, HQ), jnp.float32),
            pltpu.VMEM((SQ, DM), jnp.float32),
            pltpu.SemaphoreType.DMA((N_DEV,)),
            pltpu.SemaphoreType.DMA((N_DEV,)),
            pltpu.SemaphoreType.DMA((N_DEV,)),
            pltpu.SemaphoreType.DMA((N_DEV,)),
            pltpu.SemaphoreType.DMA((N_DEV,)),
            pltpu.SemaphoreType.DMA((N_DEV,)),
        ],
        compiler_params=pltpu.CompilerParams(
            collective_id=0,
            vmem_limit_bytes=62 * 1024 * 1024,
        ),
    )(xs, Wq, Wo, K, V)
    return out[None]
